# baseline (speedup 1.0000x reference)
import functools
import jax
import jax.numpy as jnp
from jax import lax
from jax.experimental import pallas as pl
from jax.experimental.pallas import tpu as pltpu
from jax.experimental.pallas import tpu_sc as plsc

_B, _C, _H, _W = 32, 768, 32, 32
_HW = _H * _W
_P = 32
_DIM = 384
_PAD = 40
_NJ = _C // 16
_CL = _C // 128
_NR = 1 + _HW


def _sc_body(feats_ref, pe_x_ref, pe_y_ref, glb_ref, out_ref,
             slab_a, slab_b, out_a, out_b, px_buf, row_buf, glb_buf,
             sem_ia, sem_ib, sem_oa, sem_ob):
    cid = lax.axis_index("c")
    sid = lax.axis_index("s")
    wid = sid * 2 + cid
    g = wid // 4
    u0 = 32 * (wid % 4)
    t0 = wid * _P

    pltpu.sync_copy(pe_x_ref, px_buf)
    pltpu.sync_copy(pe_y_ref.at[pl.ds(3 * wid, 3)], row_buf)
    pltpu.sync_copy(glb_ref, glb_buf)
    pltpu.sync_copy(glb_buf, out_ref.at[pl.ds(_CL * _NR * wid, _CL)])

    rowv = [row_buf[i // 8, pl.ds(16 * (i % 8), 16)] for i in range(_DIM // 16)]
    iota = lax.iota(jnp.int32, 16)
    v_lane = iota * _PAD
    zeros = jnp.zeros((16,), jnp.int32)

    def in_src(b):
        return feats_ref.at[pl.ds(b * _C, _C), g, pl.ds(u0, _P)]

    def out_dst(b):
        return out_ref.at[pl.ds(_CL * (b * _NR + 1 + t0), _CL * _P)]

    def compute(slab, out_buf):
        @plsc.parallel_loop(0, _P, unroll=2)
        def per_t(t):
            vt = v_lane + t
            for j in range(_NJ):
                idx = vt + (16 * _PAD) * j
                v = plsc.load_gather(slab, [zeros, idx])
                if j < _DIM // 16:
                    p = px_buf[3 * t + j // 8, pl.ds(16 * (j % 8), 16)]
                else:
                    p = rowv[j - _DIM // 16]
                out_buf[_CL * t + j // 8, pl.ds(16 * (j % 8), 16)] = v + p

    pltpu.async_copy(in_src(0), slab_a.at[:, pl.ds(0, _P)], sem_ia)

    def pair(i, carry):
        b_a = 2 * i
        b_b = b_a + 1

        @pl.when(i > 0)
        def _():
            pltpu.make_async_copy(out_a, out_dst(0), sem_oa).wait()
        pltpu.make_async_copy(in_src(0), slab_a.at[:, pl.ds(0, _P)],
                              sem_ia).wait()
        pltpu.async_copy(in_src(b_b), slab_b.at[:, pl.ds(0, _P)], sem_ib)
        compute(slab_a, out_a)
        pltpu.async_copy(out_a, out_dst(b_a), sem_oa)

        @pl.when(i > 0)
        def _():
            pltpu.make_async_copy(out_b, out_dst(0), sem_ob).wait()
        pltpu.make_async_copy(in_src(0), slab_b.at[:, pl.ds(0, _P)],
                              sem_ib).wait()

        @pl.when(i < _B // 2 - 1)
        def _():
            pltpu.async_copy(in_src(b_b + 1), slab_a.at[:, pl.ds(0, _P)],
                             sem_ia)
        compute(slab_b, out_b)
        pltpu.async_copy(out_b, out_dst(b_b), sem_ob)
        return carry

    lax.fori_loop(0, _B // 2, pair, 0)
    pltpu.make_async_copy(out_a, out_dst(0), sem_oa).wait()
    pltpu.make_async_copy(out_b, out_dst(0), sem_ob).wait()


def kernel(feats, pe_x_table, pe_y_table, glb_table):
    b, c, h, w = feats.shape
    hw = h * w
    feats3 = feats.reshape(b * c, hw // 128, 128)
    pe_x2 = pe_x_table.reshape(w * 3, 128)
    pe_y2 = pe_y_table.reshape(h * 3, 128)
    glb2 = glb_table.reshape(c // 128, 128)

    mesh = plsc.VectorSubcoreMesh(core_axis_name="c", subcore_axis_name="s")
    k = functools.partial(
        pl.kernel,
        mesh=mesh,
        compiler_params=pltpu.CompilerParams(
            use_tc_tiling_on_sc=False, needs_layout_passes=False),
        out_type=jax.ShapeDtypeStruct((b * (1 + hw) * (c // 128), 128),
                                      feats.dtype),
        scratch_types=[
            pltpu.VMEM((_C, _PAD), jnp.float32),
            pltpu.VMEM((_C, _PAD), jnp.float32),
            pltpu.VMEM((_CL * _P, 128), jnp.float32),
            pltpu.VMEM((_CL * _P, 128), jnp.float32),
            pltpu.VMEM((_W * 3, 128), jnp.float32),
            pltpu.VMEM((3, 128), jnp.float32),
            pltpu.VMEM((_CL, 128), jnp.float32),
            pltpu.SemaphoreType.DMA,
            pltpu.SemaphoreType.DMA,
            pltpu.SemaphoreType.DMA,
            pltpu.SemaphoreType.DMA,
        ],
    )(_sc_body)
    out2 = k(feats3, pe_x2, pe_y2, glb2)
    return out2.reshape(b, 1 + hw, c)


# speedup vs baseline: 2.3677x; 2.3677x over previous
import functools
import jax
import jax.numpy as jnp
from jax import lax
from jax.experimental import pallas as pl
from jax.experimental.pallas import tpu as pltpu
from jax.experimental.pallas import tpu_sc as plsc

_C, _H, _W = 768, 32, 32
_HW = _H * _W
_P = 32
_DIM = 384


def _peg_body(pe_x_ref, pe_y_ref, glb_ref, peg_ref, buf, row_buf, glb_buf):
    cid = lax.axis_index("c")
    sid = lax.axis_index("s")
    wid = sid * 2 + cid

    pltpu.sync_copy(pe_x_ref, buf.at[:, pl.ds(0, _DIM)])
    pltpu.sync_copy(pe_y_ref.at[wid], row_buf)

    rowv = [row_buf[pl.ds(16 * i, 16)] for i in range(_DIM // 16)]

    def fill(r, carry):
        for i in range(_DIM // 16):
            buf[r, pl.ds(_DIM + 16 * i, 16)] = rowv[i]
        return carry
    lax.fori_loop(0, _P, fill, 0)

    @pl.when(wid == 0)
    def _():
        pltpu.sync_copy(glb_ref, glb_buf)
        pltpu.sync_copy(glb_buf, peg_ref.at[pl.ds(0, 1)])

    pltpu.sync_copy(buf, peg_ref.at[pl.ds(1 + _P * wid, _P)])


def _peg_on_sc(pe_x_table, pe_y_table, glb_table):
    mesh = plsc.VectorSubcoreMesh(core_axis_name="c", subcore_axis_name="s")
    k = functools.partial(
        pl.kernel,
        mesh=mesh,
        compiler_params=pltpu.CompilerParams(
            use_tc_tiling_on_sc=False, needs_layout_passes=False),
        out_type=jax.ShapeDtypeStruct((1 + _HW, _C), jnp.float32),
        scratch_types=[
            pltpu.VMEM((_P, _C), jnp.float32),
            pltpu.VMEM((_DIM,), jnp.float32),
            pltpu.VMEM((1, _C), jnp.float32),
        ],
    )(_peg_body)
    return k(pe_x_table, pe_y_table, glb_table)


def _dense_body(feats_ref, peg_ref, out_ref):
    x = feats_ref[0]
    xt = jnp.transpose(x, (1, 0))
    out_ref[0, 1:, :] = xt + peg_ref[1:, :]
    out_ref[0, 0:1, :] = peg_ref[0:1, :]


def kernel(feats, pe_x_table, pe_y_table, glb_table):
    b, c, h, w = feats.shape
    hw = h * w
    feats2 = feats.reshape(b, c, hw)

    peg = _peg_on_sc(pe_x_table, pe_y_table, glb_table)

    out = pl.pallas_call(
        _dense_body,
        grid=(b,),
        in_specs=[
            pl.BlockSpec((1, c, hw), lambda i: (i, 0, 0)),
            pl.BlockSpec((1 + hw, c), lambda i: (0, 0)),
        ],
        out_specs=pl.BlockSpec((1, 1 + hw, c), lambda i: (i, 0, 0)),
        out_shape=jax.ShapeDtypeStruct((b, 1 + hw, c), feats.dtype),
    )(feats2, peg)
    return out
